# baseline (speedup 1.0000x reference)
import functools

import jax
import jax.numpy as jnp
from jax import lax
from jax.experimental import pallas as pl
from jax.experimental.pallas import tpu as pltpu
from jax.experimental.pallas import tpu_sc as plsc

_NC = 2
_NS = 16
_L = 16


def _lane_permute(v, idx):
    dnums = lax.GatherDimensionNumbers(
        offset_dims=(), collapsed_slice_dims=(0,), start_index_map=(0,))
    return lax.gather(v, idx[:, None], dnums, slice_sizes=(1,),
                      mode=lax.GatherScatterMode.PROMISE_IN_BOUNDS)


def _all_binop(v, op, lanes):
    for shift in (8, 4, 2, 1):
        v = op(v, _lane_permute(v, (lanes + shift) & (_L - 1)))
    return v


def _router_sc_body(x_hbm, gw_hbm, sgw_hbm, out_hbm, xv, gwv, sgwv, maskv):
    n_e = gwv.shape[0]
    d = xv.shape[1]
    n_chunks = d // _L
    wid = lax.axis_index("s") * _NC + lax.axis_index("c")
    base = wid * 2
    pltpu.sync_copy(x_hbm.at[pl.ds(base, 2)], xv)
    pltpu.sync_copy(gw_hbm, gwv)
    pltpu.sync_copy(sgw_hbm, sgwv)
    lanes = jnp.arange(_L, dtype=jnp.int32)
    neg_inf = jnp.float32(-jnp.inf)
    for t in range(2):
        def body(c, accs):
            off = pl.multiple_of(c * _L, _L)
            xa = xv[t, pl.ds(off, _L)]
            new = [accs[e] + xa * gwv[e, pl.ds(off, _L)] for e in range(n_e)]
            new.append(accs[n_e] + xa * sgwv[0, pl.ds(off, _L)])
            return tuple(new)

        accs = lax.fori_loop(
            0, n_chunks, body,
            tuple(jnp.zeros((_L,), jnp.float32) for _ in range(n_e + 1)))
        svec = jnp.full((_L,), neg_inf, dtype=jnp.float32)
        for e in range(n_e):
            tot = _all_binop(accs[e], jnp.add, lanes)
            svec = jnp.where(lanes == e, tot, svec)
        zv = _all_binop(accs[n_e], jnp.add, lanes)
        m1 = _all_binop(svec, jnp.maximum, lanes)
        a1 = _all_binop(jnp.where(svec == m1, lanes, _L), jnp.minimum, lanes)
        svec2 = jnp.where(lanes == a1, neg_inf, svec)
        m2 = _all_binop(svec2, jnp.maximum, lanes)
        a2 = _all_binop(jnp.where(svec2 == m2, lanes, _L), jnp.minimum, lanes)
        w1 = 1.0 / (1.0 + jnp.exp(m2 - m1))
        sg = 1.0 / (1.0 + jnp.exp(-zv))
        row = (jnp.where(lanes == a1, w1, 0.0)
               + jnp.where(lanes == a2, 1.0 - w1, 0.0)
               + jnp.where(lanes == n_e, sg, 0.0))
        maskv[t, :] = row
    pltpu.sync_copy(maskv, out_hbm.at[pl.ds(base, 2)])


def _router_sc(x_flat, gate_w, shared_gate_w):
    N, D = x_flat.shape
    E = gate_w.shape[0]
    mesh = plsc.VectorSubcoreMesh(core_axis_name="c", subcore_axis_name="s")
    return pl.kernel(
        _router_sc_body,
        out_type=jax.ShapeDtypeStruct((N, _L), jnp.float32),
        mesh=mesh,
        scratch_types=[
            pltpu.VMEM((2, D), jnp.float32),
            pltpu.VMEM((E, D), jnp.float32),
            pltpu.VMEM((1, D), jnp.float32),
            pltpu.VMEM((2, _L), jnp.float32),
        ],
    )(x_flat, gate_w, shared_gate_w)


def _dotTb(a, b):
    return lax.dot_general(a.astype(jnp.bfloat16), b.astype(jnp.bfloat16),
                           (((1,), (1,)), ((), ())),
                           preferred_element_type=jnp.float32)


def _moe_kernel(x_ref, msg_ref,
                wga_ref, wgb_ref, wua_ref, wub_ref, wda_ref, wdb_ref,
                w1a_ref, w1b_ref, w3a_ref, w3b_ref, w2a_ref, w2b_ref,
                out_ref, *, n_e, dh):
    i = pl.program_id(0)
    x = x_ref[...]
    xa = x[:, :dh]
    xb = x[:, dh:]
    msg = msg_ref[...]
    col = lax.broadcasted_iota(jnp.int32, msg.shape, 1)

    @pl.when(i == 0)
    def _init():
        out_ref[...] = jnp.zeros_like(out_ref)

    @pl.when(i < n_e)
    def _expert():
        me = jnp.sum(jnp.where(col == i, msg, 0.0), axis=1, keepdims=True)
        g = _dotTb(xa, wga_ref[0]) + _dotTb(xb, wgb_ref[0])
        u = _dotTb(xa, wua_ref[0]) + _dotTb(xb, wub_ref[0])
        h = jax.nn.silu(g) * u * me
        out_ref[:, :dh] += _dotTb(h, wda_ref[0])
        out_ref[:, dh:] += _dotTb(h, wdb_ref[0])

    @pl.when(i >= n_e)
    def _shared():
        sg = jnp.sum(jnp.where(col == n_e, msg, 0.0), axis=1, keepdims=True)
        s1 = _dotTb(xa, w1a_ref[...]) + _dotTb(xb, w1b_ref[...])
        s3 = _dotTb(xa, w3a_ref[...]) + _dotTb(xb, w3b_ref[...])
        sh = jax.nn.silu(s1) * s3
        out_ref[:, :dh] += sg * _dotTb(sh, w2a_ref[...])
        out_ref[:, dh:] += sg * _dotTb(sh, w2b_ref[...])


def kernel(x, gate_w, w_gate, w_up, w_down, mlp_w1, mlp_w3, mlp_w2, shared_gate_w):
    B, T, D = x.shape
    E, H, _ = w_gate.shape
    HS = mlp_w1.shape[0]
    N = B * T
    dh = D // 2
    x_flat = x.reshape(N, D)
    n_s = 11
    HSc = HS // n_s
    steps = E + n_s

    msg = _router_sc(x_flat, gate_w, shared_gate_w)

    def e_idx(i):
        return jnp.minimum(i, E - 1)

    def s_idx(i):
        return jnp.maximum(i - E, 0)

    out = pl.pallas_call(
        functools.partial(_moe_kernel, n_e=E, dh=dh),
        grid=(steps,),
        in_specs=[
            pl.BlockSpec((N, D), lambda i: (0, 0)),
            pl.BlockSpec((N, _L), lambda i: (0, 0)),
            pl.BlockSpec((1, H, dh), lambda i: (e_idx(i), 0, 0)),
            pl.BlockSpec((1, H, dh), lambda i: (e_idx(i), 0, 1)),
            pl.BlockSpec((1, H, dh), lambda i: (e_idx(i), 0, 0)),
            pl.BlockSpec((1, H, dh), lambda i: (e_idx(i), 0, 1)),
            pl.BlockSpec((1, dh, H), lambda i: (e_idx(i), 0, 0)),
            pl.BlockSpec((1, dh, H), lambda i: (e_idx(i), 1, 0)),
            pl.BlockSpec((HSc, dh), lambda i: (s_idx(i), 0)),
            pl.BlockSpec((HSc, dh), lambda i: (s_idx(i), 1)),
            pl.BlockSpec((HSc, dh), lambda i: (s_idx(i), 0)),
            pl.BlockSpec((HSc, dh), lambda i: (s_idx(i), 1)),
            pl.BlockSpec((dh, HSc), lambda i: (0, s_idx(i))),
            pl.BlockSpec((dh, HSc), lambda i: (1, s_idx(i))),
        ],
        out_specs=pl.BlockSpec((N, D), lambda i: (0, 0)),
        out_shape=jax.ShapeDtypeStruct((N, D), jnp.float32),
    )(x_flat, msg,
      w_gate, w_gate, w_up, w_up, w_down, w_down,
      mlp_w1, mlp_w1, mlp_w3, mlp_w3, mlp_w2, mlp_w2)
    return out.reshape(B, T, D)


# speedup vs baseline: 1.0602x; 1.0602x over previous
import functools

import jax
import jax.numpy as jnp
from jax import lax
from jax.experimental import pallas as pl
from jax.experimental.pallas import tpu as pltpu
from jax.experimental.pallas import tpu_sc as plsc

_NC = 2
_NS = 16
_L = 16


def _lane_permute(v, idx):
    dnums = lax.GatherDimensionNumbers(
        offset_dims=(), collapsed_slice_dims=(0,), start_index_map=(0,))
    return lax.gather(v, idx[:, None], dnums, slice_sizes=(1,),
                      mode=lax.GatherScatterMode.PROMISE_IN_BOUNDS)


def _all_binop(v, op, lanes):
    for shift in (8, 4, 2, 1):
        v = op(v, _lane_permute(v, (lanes + shift) & (_L - 1)))
    return v


def _router_sc_body(x_hbm, gw_hbm, sgw_hbm, out_hbm, xv, gwv, sgwv, maskv):
    n_e = gwv.shape[0]
    d = xv.shape[1]
    n_chunks = d // _L
    wid = lax.axis_index("s") * _NC + lax.axis_index("c")
    base = wid * 2
    pltpu.sync_copy(x_hbm.at[pl.ds(base, 2)], xv)
    pltpu.sync_copy(gw_hbm, gwv)
    pltpu.sync_copy(sgw_hbm, sgwv)
    lanes = jnp.arange(_L, dtype=jnp.int32)
    neg_inf = jnp.float32(-jnp.inf)
    for t in range(2):
        def body(c, accs):
            off = pl.multiple_of(c * _L, _L)
            xa = xv[t, pl.ds(off, _L)]
            new = [accs[e] + xa * gwv[e, pl.ds(off, _L)] for e in range(n_e)]
            new.append(accs[n_e] + xa * sgwv[0, pl.ds(off, _L)])
            return tuple(new)

        accs = lax.fori_loop(
            0, n_chunks, body,
            tuple(jnp.zeros((_L,), jnp.float32) for _ in range(n_e + 1)))
        svec = jnp.full((_L,), neg_inf, dtype=jnp.float32)
        for e in range(n_e):
            tot = _all_binop(accs[e], jnp.add, lanes)
            svec = jnp.where(lanes == e, tot, svec)
        zv = _all_binop(accs[n_e], jnp.add, lanes)
        m1 = _all_binop(svec, jnp.maximum, lanes)
        a1 = _all_binop(jnp.where(svec == m1, lanes, _L), jnp.minimum, lanes)
        svec2 = jnp.where(lanes == a1, neg_inf, svec)
        m2 = _all_binop(svec2, jnp.maximum, lanes)
        a2 = _all_binop(jnp.where(svec2 == m2, lanes, _L), jnp.minimum, lanes)
        w1 = 1.0 / (1.0 + jnp.exp(m2 - m1))
        sg = 1.0 / (1.0 + jnp.exp(-zv))
        row = (jnp.where(lanes == a1, w1, 0.0)
               + jnp.where(lanes == a2, 1.0 - w1, 0.0)
               + jnp.where(lanes == n_e, sg, 0.0))
        maskv[t, :] = row
    pltpu.sync_copy(maskv, out_hbm.at[pl.ds(base, 2)])


def _router_sc(x_flat, gate_w, shared_gate_w):
    N, D = x_flat.shape
    E = gate_w.shape[0]
    mesh = plsc.VectorSubcoreMesh(core_axis_name="c", subcore_axis_name="s")
    return pl.kernel(
        _router_sc_body,
        out_type=jax.ShapeDtypeStruct((N, _L), jnp.float32),
        mesh=mesh,
        scratch_types=[
            pltpu.VMEM((2, D), jnp.float32),
            pltpu.VMEM((E, D), jnp.float32),
            pltpu.VMEM((1, D), jnp.float32),
            pltpu.VMEM((2, _L), jnp.float32),
        ],
    )(x_flat, gate_w, shared_gate_w)


def _dotTb(a, b):
    return lax.dot_general(a.astype(jnp.bfloat16), b.astype(jnp.bfloat16),
                           (((1,), (1,)), ((), ())),
                           preferred_element_type=jnp.float32)


def _shared_kernel(x_ref, w1a_ref, w1b_ref, w3a_ref, w3b_ref,
                   w2a_ref, w2b_ref, out_ref, *, dh):
    j = pl.program_id(0)
    x = x_ref[...]
    xa = x[:, :dh]
    xb = x[:, dh:]

    @pl.when(j == 0)
    def _init():
        out_ref[...] = jnp.zeros_like(out_ref)

    s1 = _dotTb(xa, w1a_ref[...]) + _dotTb(xb, w1b_ref[...])
    s3 = _dotTb(xa, w3a_ref[...]) + _dotTb(xb, w3b_ref[...])
    sh = jax.nn.silu(s1) * s3
    out_ref[:, :dh] += _dotTb(sh, w2a_ref[...])
    out_ref[:, dh:] += _dotTb(sh, w2b_ref[...])


def _expert_kernel(x_ref, msg_ref, ssum_ref,
                   wga_ref, wgb_ref, wua_ref, wub_ref, wda_ref, wdb_ref,
                   out_ref, *, n_e, dh):
    i = pl.program_id(0)
    x = x_ref[...]
    xa = x[:, :dh]
    xb = x[:, dh:]
    msg = msg_ref[...]
    col = lax.broadcasted_iota(jnp.int32, msg.shape, 1)

    @pl.when(i == 0)
    def _init():
        sg = jnp.sum(jnp.where(col == n_e, msg, 0.0), axis=1, keepdims=True)
        out_ref[...] = sg * ssum_ref[...]

    me = jnp.sum(jnp.where(col == i, msg, 0.0), axis=1, keepdims=True)
    g = _dotTb(xa, wga_ref[0]) + _dotTb(xb, wgb_ref[0])
    u = _dotTb(xa, wua_ref[0]) + _dotTb(xb, wub_ref[0])
    h = jax.nn.silu(g) * u * me
    out_ref[:, :dh] += _dotTb(h, wda_ref[0])
    out_ref[:, dh:] += _dotTb(h, wdb_ref[0])


def kernel(x, gate_w, w_gate, w_up, w_down, mlp_w1, mlp_w3, mlp_w2, shared_gate_w):
    B, T, D = x.shape
    E, H, _ = w_gate.shape
    HS = mlp_w1.shape[0]
    N = B * T
    dh = D // 2
    x_flat = x.reshape(N, D)
    n_s = 11
    HSc = HS // n_s

    msg = _router_sc(x_flat, gate_w, shared_gate_w)

    ssum = pl.pallas_call(
        functools.partial(_shared_kernel, dh=dh),
        grid=(n_s,),
        in_specs=[
            pl.BlockSpec((N, D), lambda j: (0, 0)),
            pl.BlockSpec((HSc, dh), lambda j: (j, 0)),
            pl.BlockSpec((HSc, dh), lambda j: (j, 1)),
            pl.BlockSpec((HSc, dh), lambda j: (j, 0)),
            pl.BlockSpec((HSc, dh), lambda j: (j, 1)),
            pl.BlockSpec((dh, HSc), lambda j: (0, j)),
            pl.BlockSpec((dh, HSc), lambda j: (1, j)),
        ],
        out_specs=pl.BlockSpec((N, D), lambda j: (0, 0)),
        out_shape=jax.ShapeDtypeStruct((N, D), jnp.float32),
    )(x_flat, mlp_w1, mlp_w1, mlp_w3, mlp_w3, mlp_w2, mlp_w2)

    out = pl.pallas_call(
        functools.partial(_expert_kernel, n_e=E, dh=dh),
        grid=(E,),
        in_specs=[
            pl.BlockSpec((N, D), lambda e: (0, 0)),
            pl.BlockSpec((N, _L), lambda e: (0, 0)),
            pl.BlockSpec((N, D), lambda e: (0, 0)),
            pl.BlockSpec((1, H, dh), lambda e: (e, 0, 0)),
            pl.BlockSpec((1, H, dh), lambda e: (e, 0, 1)),
            pl.BlockSpec((1, H, dh), lambda e: (e, 0, 0)),
            pl.BlockSpec((1, H, dh), lambda e: (e, 0, 1)),
            pl.BlockSpec((1, dh, H), lambda e: (e, 0, 0)),
            pl.BlockSpec((1, dh, H), lambda e: (e, 1, 0)),
        ],
        out_specs=pl.BlockSpec((N, D), lambda e: (0, 0)),
        out_shape=jax.ShapeDtypeStruct((N, D), jnp.float32),
    )(x_flat, msg, ssum,
      w_gate, w_gate, w_up, w_up, w_down, w_down)
    return out.reshape(B, T, D)


# speedup vs baseline: 1.1035x; 1.0408x over previous
import functools

import jax
import jax.numpy as jnp
from jax import lax
from jax.experimental import pallas as pl
from jax.experimental.pallas import tpu as pltpu
from jax.experimental.pallas import tpu_sc as plsc

_NC = 2
_NS = 16
_L = 16


def _lane_permute(v, idx):
    dnums = lax.GatherDimensionNumbers(
        offset_dims=(), collapsed_slice_dims=(0,), start_index_map=(0,))
    return lax.gather(v, idx[:, None], dnums, slice_sizes=(1,),
                      mode=lax.GatherScatterMode.PROMISE_IN_BOUNDS)


def _all_binop(v, op, lanes):
    for shift in (8, 4, 2, 1):
        v = op(v, _lane_permute(v, (lanes + shift) & (_L - 1)))
    return v


def _router_sc_body(x_hbm, gw_hbm, sgw_hbm, out_hbm, xv, gwv, sgwv, maskv, gw_sh, sgw_sh):
    n_e = gwv.shape[0]
    d = xv.shape[1]
    n_chunks = d // _L
    sid = lax.axis_index("s")
    wid = sid * _NC + lax.axis_index("c")
    base = wid * 2
    pltpu.sync_copy(x_hbm.at[pl.ds(base, 2)], xv)
    @pl.when(sid == 0)
    def _stage():
        pltpu.sync_copy(gw_hbm, gwv)
        pltpu.sync_copy(sgw_hbm, sgwv)
        pltpu.sync_copy(gwv, gw_sh)
        pltpu.sync_copy(sgwv, sgw_sh)
    plsc.subcore_barrier()
    @pl.when(sid != 0)
    def _fetch():
        pltpu.sync_copy(gw_sh, gwv)
        pltpu.sync_copy(sgw_sh, sgwv)
    lanes = jnp.arange(_L, dtype=jnp.int32)
    neg_inf = jnp.float32(-jnp.inf)
    for t in range(2):
        def body(c, accs):
            off = pl.multiple_of(c * _L, _L)
            xa = xv[t, pl.ds(off, _L)]
            new = [accs[e] + xa * gwv[e, pl.ds(off, _L)] for e in range(n_e)]
            new.append(accs[n_e] + xa * sgwv[0, pl.ds(off, _L)])
            return tuple(new)

        accs = lax.fori_loop(
            0, n_chunks, body,
            tuple(jnp.zeros((_L,), jnp.float32) for _ in range(n_e + 1)))
        svec = jnp.full((_L,), neg_inf, dtype=jnp.float32)
        for e in range(n_e):
            tot = _all_binop(accs[e], jnp.add, lanes)
            svec = jnp.where(lanes == e, tot, svec)
        zv = _all_binop(accs[n_e], jnp.add, lanes)
        m1 = _all_binop(svec, jnp.maximum, lanes)
        a1 = _all_binop(jnp.where(svec == m1, lanes, _L), jnp.minimum, lanes)
        svec2 = jnp.where(lanes == a1, neg_inf, svec)
        m2 = _all_binop(svec2, jnp.maximum, lanes)
        a2 = _all_binop(jnp.where(svec2 == m2, lanes, _L), jnp.minimum, lanes)
        w1 = 1.0 / (1.0 + jnp.exp(m2 - m1))
        sg = 1.0 / (1.0 + jnp.exp(-zv))
        row = (jnp.where(lanes == a1, w1, 0.0)
               + jnp.where(lanes == a2, 1.0 - w1, 0.0)
               + jnp.where(lanes == n_e, sg, 0.0))
        maskv[t, :] = row
    pltpu.sync_copy(maskv, out_hbm.at[pl.ds(base, 2)])


def _router_sc(x_flat, gate_w, shared_gate_w):
    N, D = x_flat.shape
    E = gate_w.shape[0]
    mesh = plsc.VectorSubcoreMesh(core_axis_name="c", subcore_axis_name="s")
    return pl.kernel(
        _router_sc_body,
        out_type=jax.ShapeDtypeStruct((N, _L), jnp.float32),
        mesh=mesh,
        scratch_types=[
            pltpu.VMEM((2, D), jnp.float32),
            pltpu.VMEM((E, D), jnp.float32),
            pltpu.VMEM((1, D), jnp.float32),
            pltpu.VMEM((2, _L), jnp.float32),
            pltpu.VMEM_SHARED((E, D), jnp.float32),
            pltpu.VMEM_SHARED((1, D), jnp.float32),
        ],
    )(x_flat, gate_w, shared_gate_w)


def _dotTb(a, b):
    return lax.dot_general(a.astype(jnp.bfloat16), b.astype(jnp.bfloat16),
                           (((1,), (1,)), ((), ())),
                           preferred_element_type=jnp.float32)


def _shared_kernel(x_ref, w1a_ref, w1b_ref, w3a_ref, w3b_ref,
                   w2a_ref, w2b_ref, out_ref, *, dh):
    j = pl.program_id(0)
    x = x_ref[...]
    xa = x[:, :dh]
    xb = x[:, dh:]

    @pl.when(j == 0)
    def _init():
        out_ref[...] = jnp.zeros_like(out_ref)

    s1 = _dotTb(xa, w1a_ref[...]) + _dotTb(xb, w1b_ref[...])
    s3 = _dotTb(xa, w3a_ref[...]) + _dotTb(xb, w3b_ref[...])
    sh = jax.nn.silu(s1) * s3
    out_ref[:, :dh] += _dotTb(sh, w2a_ref[...])
    out_ref[:, dh:] += _dotTb(sh, w2b_ref[...])


def _expert_kernel(x_ref, msg_ref, ssum_ref,
                   wga_ref, wgb_ref, wua_ref, wub_ref, wda_ref, wdb_ref,
                   out_ref, *, n_e, dh):
    i = pl.program_id(0)
    x = x_ref[...]
    xa = x[:, :dh]
    xb = x[:, dh:]
    msg = msg_ref[...]
    col = lax.broadcasted_iota(jnp.int32, msg.shape, 1)

    @pl.when(i == 0)
    def _init():
        sg = jnp.sum(jnp.where(col == n_e, msg, 0.0), axis=1, keepdims=True)
        out_ref[...] = sg * ssum_ref[...]

    me = jnp.sum(jnp.where(col == i, msg, 0.0), axis=1, keepdims=True)
    g = _dotTb(xa, wga_ref[0]) + _dotTb(xb, wgb_ref[0])
    u = _dotTb(xa, wua_ref[0]) + _dotTb(xb, wub_ref[0])
    h = jax.nn.silu(g) * u * me
    out_ref[:, :dh] += _dotTb(h, wda_ref[0])
    out_ref[:, dh:] += _dotTb(h, wdb_ref[0])


def kernel(x, gate_w, w_gate, w_up, w_down, mlp_w1, mlp_w3, mlp_w2, shared_gate_w):
    B, T, D = x.shape
    E, H, _ = w_gate.shape
    HS = mlp_w1.shape[0]
    N = B * T
    dh = D // 2
    x_flat = x.reshape(N, D)
    n_s = 11
    HSc = HS // n_s

    msg = _router_sc(x_flat, gate_w, shared_gate_w)

    ssum = pl.pallas_call(
        functools.partial(_shared_kernel, dh=dh),
        grid=(n_s,),
        in_specs=[
            pl.BlockSpec((N, D), lambda j: (0, 0)),
            pl.BlockSpec((HSc, dh), lambda j: (j, 0)),
            pl.BlockSpec((HSc, dh), lambda j: (j, 1)),
            pl.BlockSpec((HSc, dh), lambda j: (j, 0)),
            pl.BlockSpec((HSc, dh), lambda j: (j, 1)),
            pl.BlockSpec((dh, HSc), lambda j: (0, j)),
            pl.BlockSpec((dh, HSc), lambda j: (1, j)),
        ],
        out_specs=pl.BlockSpec((N, D), lambda j: (0, 0)),
        out_shape=jax.ShapeDtypeStruct((N, D), jnp.float32),
    )(x_flat, mlp_w1, mlp_w1, mlp_w3, mlp_w3, mlp_w2, mlp_w2)

    out = pl.pallas_call(
        functools.partial(_expert_kernel, n_e=E, dh=dh),
        grid=(E,),
        in_specs=[
            pl.BlockSpec((N, D), lambda e: (0, 0)),
            pl.BlockSpec((N, _L), lambda e: (0, 0)),
            pl.BlockSpec((N, D), lambda e: (0, 0)),
            pl.BlockSpec((1, H, dh), lambda e: (e, 0, 0)),
            pl.BlockSpec((1, H, dh), lambda e: (e, 0, 1)),
            pl.BlockSpec((1, H, dh), lambda e: (e, 0, 0)),
            pl.BlockSpec((1, H, dh), lambda e: (e, 0, 1)),
            pl.BlockSpec((1, dh, H), lambda e: (e, 0, 0)),
            pl.BlockSpec((1, dh, H), lambda e: (e, 1, 0)),
        ],
        out_specs=pl.BlockSpec((N, D), lambda e: (0, 0)),
        out_shape=jax.ShapeDtypeStruct((N, D), jnp.float32),
    )(x_flat, msg, ssum,
      w_gate, w_gate, w_up, w_up, w_down, w_down)
    return out.reshape(B, T, D)


# speedup vs baseline: 1.1199x; 1.0149x over previous
import functools

import jax
import jax.numpy as jnp
from jax import lax
from jax.experimental import pallas as pl
from jax.experimental.pallas import tpu as pltpu
from jax.experimental.pallas import tpu_sc as plsc

_NC = 2
_NS = 16
_L = 16


def _lane_permute(v, idx):
    dnums = lax.GatherDimensionNumbers(
        offset_dims=(), collapsed_slice_dims=(0,), start_index_map=(0,))
    return lax.gather(v, idx[:, None], dnums, slice_sizes=(1,),
                      mode=lax.GatherScatterMode.PROMISE_IN_BOUNDS)


def _all_binop(v, op, lanes):
    for shift in (8, 4, 2, 1):
        v = op(v, _lane_permute(v, (lanes + shift) & (_L - 1)))
    return v


def _router_sc_body(x_hbm, gw_hbm, sgw_hbm, out_hbm, xv, gwv, sgwv, maskv, gw_sh, sgw_sh):
    n_e = gwv.shape[0]
    d = xv.shape[1]
    n_chunks = d // _L
    sid = lax.axis_index("s")
    wid = sid
    base = wid * 4
    pltpu.sync_copy(x_hbm.at[pl.ds(base, 4)], xv)
    @pl.when(sid == 0)
    def _stage():
        pltpu.sync_copy(gw_hbm, gwv)
        pltpu.sync_copy(sgw_hbm, sgwv)
        pltpu.sync_copy(gwv, gw_sh)
        pltpu.sync_copy(sgwv, sgw_sh)
    plsc.subcore_barrier()
    @pl.when(sid != 0)
    def _fetch():
        pltpu.sync_copy(gw_sh, gwv)
        pltpu.sync_copy(sgw_sh, sgwv)
    lanes = jnp.arange(_L, dtype=jnp.int32)
    neg_inf = jnp.float32(-jnp.inf)
    for t in range(4):
        def body(c, accs):
            off = pl.multiple_of(c * _L, _L)
            xa = xv[t, pl.ds(off, _L)]
            new = [accs[e] + xa * gwv[e, pl.ds(off, _L)] for e in range(n_e)]
            new.append(accs[n_e] + xa * sgwv[0, pl.ds(off, _L)])
            return tuple(new)

        accs = lax.fori_loop(
            0, n_chunks, body,
            tuple(jnp.zeros((_L,), jnp.float32) for _ in range(n_e + 1)))
        svec = jnp.full((_L,), neg_inf, dtype=jnp.float32)
        for e in range(n_e):
            tot = _all_binop(accs[e], jnp.add, lanes)
            svec = jnp.where(lanes == e, tot, svec)
        zv = _all_binop(accs[n_e], jnp.add, lanes)
        m1 = _all_binop(svec, jnp.maximum, lanes)
        a1 = _all_binop(jnp.where(svec == m1, lanes, _L), jnp.minimum, lanes)
        svec2 = jnp.where(lanes == a1, neg_inf, svec)
        m2 = _all_binop(svec2, jnp.maximum, lanes)
        a2 = _all_binop(jnp.where(svec2 == m2, lanes, _L), jnp.minimum, lanes)
        w1 = 1.0 / (1.0 + jnp.exp(m2 - m1))
        sg = 1.0 / (1.0 + jnp.exp(-zv))
        row = (jnp.where(lanes == a1, w1, 0.0)
               + jnp.where(lanes == a2, 1.0 - w1, 0.0)
               + jnp.where(lanes == n_e, sg, 0.0))
        maskv[t, :] = row
    pltpu.sync_copy(maskv, out_hbm.at[pl.ds(base, 4)])


def _router_sc(x_flat, gate_w, shared_gate_w):
    N, D = x_flat.shape
    E = gate_w.shape[0]
    mesh = plsc.VectorSubcoreMesh(core_axis_name="c", subcore_axis_name="s", num_cores=1)
    return pl.kernel(
        _router_sc_body,
        out_type=jax.ShapeDtypeStruct((N, _L), jnp.float32),
        mesh=mesh,
        scratch_types=[
            pltpu.VMEM((4, D), jnp.float32),
            pltpu.VMEM((E, D), jnp.float32),
            pltpu.VMEM((1, D), jnp.float32),
            pltpu.VMEM((4, _L), jnp.float32),
            pltpu.VMEM_SHARED((E, D), jnp.float32),
            pltpu.VMEM_SHARED((1, D), jnp.float32),
        ],
    )(x_flat, gate_w, shared_gate_w)


def _dotTb(a, b):
    return lax.dot_general(a.astype(jnp.bfloat16), b.astype(jnp.bfloat16),
                           (((1,), (1,)), ((), ())),
                           preferred_element_type=jnp.float32)


def _shared_kernel(x_ref, w1a_ref, w1b_ref, w3a_ref, w3b_ref,
                   w2a_ref, w2b_ref, out_ref, *, dh):
    j = pl.program_id(0)
    x = x_ref[...]
    xa = x[:, :dh]
    xb = x[:, dh:]

    @pl.when(j == 0)
    def _init():
        out_ref[...] = jnp.zeros_like(out_ref)

    s1 = _dotTb(xa, w1a_ref[...]) + _dotTb(xb, w1b_ref[...])
    s3 = _dotTb(xa, w3a_ref[...]) + _dotTb(xb, w3b_ref[...])
    sh = jax.nn.silu(s1) * s3
    out_ref[:, :dh] += _dotTb(sh, w2a_ref[...])
    out_ref[:, dh:] += _dotTb(sh, w2b_ref[...])


def _expert_kernel(x_ref, msg_ref, ssum_ref,
                   wga_ref, wgb_ref, wua_ref, wub_ref, wda_ref, wdb_ref,
                   out_ref, *, n_e, dh):
    i = pl.program_id(0)
    x = x_ref[...]
    xa = x[:, :dh]
    xb = x[:, dh:]
    msg = msg_ref[...]
    col = lax.broadcasted_iota(jnp.int32, msg.shape, 1)

    @pl.when(i == 0)
    def _init():
        sg = jnp.sum(jnp.where(col == n_e, msg, 0.0), axis=1, keepdims=True)
        out_ref[...] = sg * ssum_ref[...]

    me = jnp.sum(jnp.where(col == i, msg, 0.0), axis=1, keepdims=True)
    g = _dotTb(xa, wga_ref[0]) + _dotTb(xb, wgb_ref[0])
    u = _dotTb(xa, wua_ref[0]) + _dotTb(xb, wub_ref[0])
    h = jax.nn.silu(g) * u * me
    out_ref[:, :dh] += _dotTb(h, wda_ref[0])
    out_ref[:, dh:] += _dotTb(h, wdb_ref[0])


def kernel(x, gate_w, w_gate, w_up, w_down, mlp_w1, mlp_w3, mlp_w2, shared_gate_w):
    B, T, D = x.shape
    E, H, _ = w_gate.shape
    HS = mlp_w1.shape[0]
    N = B * T
    dh = D // 2
    x_flat = x.reshape(N, D)
    n_s = 11
    HSc = HS // n_s

    msg = _router_sc(x_flat, gate_w, shared_gate_w)

    ssum = pl.pallas_call(
        functools.partial(_shared_kernel, dh=dh),
        grid=(n_s,),
        in_specs=[
            pl.BlockSpec((N, D), lambda j: (0, 0)),
            pl.BlockSpec((HSc, dh), lambda j: (j, 0)),
            pl.BlockSpec((HSc, dh), lambda j: (j, 1)),
            pl.BlockSpec((HSc, dh), lambda j: (j, 0)),
            pl.BlockSpec((HSc, dh), lambda j: (j, 1)),
            pl.BlockSpec((dh, HSc), lambda j: (0, j)),
            pl.BlockSpec((dh, HSc), lambda j: (1, j)),
        ],
        out_specs=pl.BlockSpec((N, D), lambda j: (0, 0)),
        out_shape=jax.ShapeDtypeStruct((N, D), jnp.float32),
    )(x_flat, mlp_w1, mlp_w1, mlp_w3, mlp_w3, mlp_w2, mlp_w2)

    out = pl.pallas_call(
        functools.partial(_expert_kernel, n_e=E, dh=dh),
        grid=(E,),
        in_specs=[
            pl.BlockSpec((N, D), lambda e: (0, 0)),
            pl.BlockSpec((N, _L), lambda e: (0, 0)),
            pl.BlockSpec((N, D), lambda e: (0, 0)),
            pl.BlockSpec((1, H, dh), lambda e: (e, 0, 0)),
            pl.BlockSpec((1, H, dh), lambda e: (e, 0, 1)),
            pl.BlockSpec((1, H, dh), lambda e: (e, 0, 0)),
            pl.BlockSpec((1, H, dh), lambda e: (e, 0, 1)),
            pl.BlockSpec((1, dh, H), lambda e: (e, 0, 0)),
            pl.BlockSpec((1, dh, H), lambda e: (e, 1, 0)),
        ],
        out_specs=pl.BlockSpec((N, D), lambda e: (0, 0)),
        out_shape=jax.ShapeDtypeStruct((N, D), jnp.float32),
    )(x_flat, msg, ssum,
      w_gate, w_gate, w_up, w_up, w_down, w_down)
    return out.reshape(B, T, D)
